# Initial kernel scaffold; baseline (speedup 1.0000x reference)
import jax
import jax.numpy as jnp
from jax.experimental import pallas as pl


def kernel(x, conv1_band, conv1_bias, pool_sl, pool_sr, conv2_band, conv2_bias, fc1_w, fc1_b, fc2_w, fc2_b):
    raise NotImplementedError("write your pallas kernel here")


# speedup vs baseline: 9.7868x; 9.7868x over previous
import functools

import jax
import jax.numpy as jnp
from jax.experimental import pallas as pl
from jax.experimental.pallas import tpu as pltpu


def _fused_kernel(x_ref, c1_ref, c1b_ref, sle_ref, sr_ref, c2_ref, c2b_ref,
                  w1_ref, b1_ref, w2_ref, b2_ref, o_ref):
    B = x_ref.shape[0]
    x = x_ref[...]

    lhs1 = jnp.concatenate([x[:, ki:ki + 24, :] for ki in range(5)],
                           axis=-1).reshape(B * 24, 140)
    y1 = jnp.dot(lhs1, c1_ref[...], preferred_element_type=jnp.float32)
    y1 = jnp.maximum(y1 + c1b_ref[...], 0.0)
    y1 = y1.reshape(B, 24, 240)

    mh = jnp.maximum(y1, jnp.concatenate([y1[:, 1:, :], y1[:, 23:, :]],
                                         axis=1))
    mhw = jnp.maximum(mh[:, :, 0:239], mh[:, :, 1:240])

    n = jnp.dot(mhw.reshape(B * 24, 239), sr_ref[...],
                preferred_element_type=jnp.float32)
    n3 = n.reshape(B, 24, 120)

    p_rows = [jnp.sum(n3 * sle_ref[i], axis=1) for i in range(12)]

    lhs2 = jnp.concatenate(
        [jnp.concatenate([p_rows[h], p_rows[h + 1], p_rows[h + 2]],
                         axis=-1)[:, None, :] for h in range(10)]
        + [jnp.zeros((B, 6, 360), jnp.float32)], axis=1)
    y2 = jnp.dot(lhs2.reshape(B * 16, 360), c2_ref[...],
                 preferred_element_type=jnp.float32)
    y2 = jnp.maximum(y2 + c2b_ref[...], 0.0)
    y2 = y2.reshape(B, 16, 200)

    acc = b1_ref[...]
    acc = acc + sum(
        jnp.dot(y2[:, h, :], w1_ref[h], preferred_element_type=jnp.float32)
        for h in range(10))
    h1 = jnp.maximum(acc, 0.0)

    z = jnp.dot(h1, w2_ref[...], preferred_element_type=jnp.float32) \
        + b2_ref[...]
    m = jnp.max(z, axis=-1, keepdims=True)
    lse = jnp.log(jnp.sum(jnp.exp(z - m), axis=-1, keepdims=True)) + m
    o_ref[...] = z - lse


@functools.partial(jax.jit, static_argnames=())
def kernel(x, conv1_band, conv1_bias, pool_sl, pool_sr, conv2_band,
           conv2_bias, fc1_w, fc1_b, fc2_w, fc2_b):
    N = x.shape[0]
    x2 = x.reshape(N, 28, 28)

    B = 64
    while N % B:
        B //= 2
    grid = N // B

    c1r = conv1_band.reshape(140, 240)
    sle = jnp.pad(pool_sl, ((0, 0), (0, 1)))[:, :, None]
    c2r = conv2_band.reshape(360, 200)
    w1r = fc1_w.reshape(10, 200, 500)

    return pl.pallas_call(
        _fused_kernel,
        out_shape=jax.ShapeDtypeStruct((N, 10), jnp.float32),
        grid=(grid,),
        in_specs=[
            pl.BlockSpec((B, 28, 28), lambda b: (b, 0, 0)),
            pl.BlockSpec((140, 240), lambda b: (0, 0)),
            pl.BlockSpec((1, 240), lambda b: (0, 0)),
            pl.BlockSpec((12, 24, 1), lambda b: (0, 0, 0)),
            pl.BlockSpec((239, 120), lambda b: (0, 0)),
            pl.BlockSpec((360, 200), lambda b: (0, 0)),
            pl.BlockSpec((1, 200), lambda b: (0, 0)),
            pl.BlockSpec((10, 200, 500), lambda b: (0, 0, 0)),
            pl.BlockSpec((1, 500), lambda b: (0, 0)),
            pl.BlockSpec((500, 10), lambda b: (0, 0)),
            pl.BlockSpec((1, 10), lambda b: (0, 0)),
        ],
        out_specs=pl.BlockSpec((B, 10), lambda b: (b, 0)),
        compiler_params=pltpu.CompilerParams(
            dimension_semantics=("parallel",)),
        cost_estimate=pl.CostEstimate(
            flops=N * (24 * 140 * 240 + 24 * 239 * 120 + 16 * 360 * 200
                       + 2000 * 500 + 500 * 10) * 2,
            transcendentals=N * 11,
            bytes_accessed=N * (784 + 10) * 4 + 4 * (140 * 240 + 239 * 120
                                                     + 360 * 200 + 2000 * 500
                                                     + 500 * 10),
        ),
    )(x2, c1r, conv1_bias, sle, pool_sr, c2r, conv2_bias, w1r, fc1_b,
      fc2_w, fc2_b)


# speedup vs baseline: 10.3591x; 1.0585x over previous
import functools

import jax
import jax.numpy as jnp
from jax.experimental import pallas as pl
from jax.experimental.pallas import tpu as pltpu


def _fused_kernel(x_ref, c1_ref, c1b_ref, sle_ref, sr_ref, c2_ref, c2b_ref,
                  w1_ref, b1_ref, w2_ref, b2_ref, o_ref):
    B = x_ref.shape[0]
    x = x_ref[...]

    xp = jnp.pad(x, ((0, 0), (0, 0), (0, 100)))
    lhs1 = jnp.concatenate([xp[:, ki:ki + 24, :] for ki in range(5)],
                           axis=-1).reshape(B * 24, 640)
    y1 = jnp.dot(lhs1, c1_ref[...], preferred_element_type=jnp.float32)
    y1 = jnp.maximum(y1 + c1b_ref[...], 0.0)
    y1 = y1.reshape(B, 24, 240)

    mh = jnp.maximum(y1, jnp.concatenate([y1[:, 1:, :], y1[:, 23:, :]],
                                         axis=1))
    mhw = jnp.maximum(mh[:, :, 0:239], mh[:, :, 1:240])

    n = jnp.dot(mhw.reshape(B * 24, 239), sr_ref[...],
                preferred_element_type=jnp.float32)
    n3 = n.reshape(B, 24, 120)

    p_hmaj = jnp.concatenate(
        [jnp.sum(n3 * sle_ref[i], axis=1) for i in range(12)],
        axis=0)

    acc2 = jnp.dot(p_hmaj[0:10 * B], c2_ref[0],
                   preferred_element_type=jnp.float32)
    acc2 = acc2 + jnp.dot(p_hmaj[B:11 * B], c2_ref[1],
                          preferred_element_type=jnp.float32)
    acc2 = acc2 + jnp.dot(p_hmaj[2 * B:12 * B], c2_ref[2],
                          preferred_element_type=jnp.float32)
    y2 = jnp.maximum(acc2 + c2b_ref[...], 0.0)

    acc = b1_ref[...] + sum(
        jnp.dot(y2[h * B:(h + 1) * B], w1_ref[h],
                preferred_element_type=jnp.float32)
        for h in range(10))
    h1 = jnp.maximum(acc, 0.0)

    z = jnp.dot(h1, w2_ref[...], preferred_element_type=jnp.float32) \
        + b2_ref[...]
    m = jnp.max(z, axis=-1, keepdims=True)
    lse = jnp.log(jnp.sum(jnp.exp(z - m), axis=-1, keepdims=True)) + m
    o_ref[...] = z - lse


@functools.partial(jax.jit, static_argnames=())
def kernel(x, conv1_band, conv1_bias, pool_sl, pool_sr, conv2_band,
           conv2_bias, fc1_w, fc1_b, fc2_w, fc2_b):
    N = x.shape[0]
    x2 = x.reshape(N, 28, 28)

    B = 64
    while N % B:
        B //= 2
    grid = N // B

    c1p = jnp.zeros((5, 128, 240), jnp.float32).at[:, :28, :].set(conv1_band)
    c1p = c1p.reshape(640, 240)
    sle = jnp.pad(pool_sl, ((0, 0), (0, 1)))[:, :, None]
    w1r = fc1_w.reshape(10, 200, 500)

    return pl.pallas_call(
        _fused_kernel,
        out_shape=jax.ShapeDtypeStruct((N, 10), jnp.float32),
        grid=(grid,),
        in_specs=[
            pl.BlockSpec((B, 28, 28), lambda b: (b, 0, 0)),
            pl.BlockSpec((640, 240), lambda b: (0, 0)),
            pl.BlockSpec((1, 240), lambda b: (0, 0)),
            pl.BlockSpec((12, 24, 1), lambda b: (0, 0, 0)),
            pl.BlockSpec((239, 120), lambda b: (0, 0)),
            pl.BlockSpec((3, 120, 200), lambda b: (0, 0, 0)),
            pl.BlockSpec((1, 200), lambda b: (0, 0)),
            pl.BlockSpec((10, 200, 500), lambda b: (0, 0, 0)),
            pl.BlockSpec((1, 500), lambda b: (0, 0)),
            pl.BlockSpec((500, 10), lambda b: (0, 0)),
            pl.BlockSpec((1, 10), lambda b: (0, 0)),
        ],
        out_specs=pl.BlockSpec((B, 10), lambda b: (b, 0)),
        compiler_params=pltpu.CompilerParams(
            dimension_semantics=("parallel",)),
        cost_estimate=pl.CostEstimate(
            flops=N * (24 * 140 * 240 + 24 * 239 * 120 + 10 * 360 * 200
                       + 2000 * 500 + 500 * 10) * 2,
            transcendentals=N * 11,
            bytes_accessed=N * (784 + 10) * 4 + 4 * (140 * 240 + 239 * 120
                                                     + 360 * 200 + 2000 * 500
                                                     + 500 * 10),
        ),
    )(x2, c1p, conv1_bias, sle, pool_sr, conv2_band, conv2_bias, w1r, fc1_b,
      fc2_w, fc2_b)
